# baseline (speedup 1.0000x reference)
import functools
import numpy as np

import jax
import jax.numpy as jnp
from jax import lax
from jax.experimental import pallas as pl
from jax.experimental.pallas import tpu as pltpu
from jax.experimental.pallas import tpu_sc as plsc

_MS0 = 1.52563191462
_MS1 = 1.62856739989
_MS2 = 3.88311640418
_LOC_SCOPE = 3.0
_LOC_BIN = 0.5
_NHB = 12
_PRE = 2048
_POST = 512
_THRESH = 0.85
_BLK = 256


def _sort_body(scores_ref, out_s_ref, out_i_ref):
    s = scores_ref[...]
    B = s.shape[0]
    row = lax.broadcasted_iota(jnp.int32, s.shape, 1)
    lane = lax.broadcasted_iota(jnp.int32, s.shape, 2)
    bat = lax.broadcasted_iota(jnp.int32, s.shape, 0)
    n = 128 * 128
    idx = (bat * n + row * 128 + lane).astype(jnp.float32)

    k = 2
    while k <= n:
        j = k // 2
        while j > 0:
            if j < 128:
                is_lo = (lane & j) == 0
                bs = jnp.where(is_lo, jnp.roll(s, -j, axis=2), jnp.roll(s, j, axis=2))
                bi = jnp.where(is_lo, jnp.roll(idx, -j, axis=2), jnp.roll(idx, j, axis=2))
            else:
                m = j // 128
                is_lo = (row & m) == 0
                bs = jnp.where(is_lo, jnp.roll(s, -m, axis=1), jnp.roll(s, m, axis=1))
                bi = jnp.where(is_lo, jnp.roll(idx, -m, axis=1), jnp.roll(idx, m, axis=1))
            if k >= n:
                sel_first = is_lo
            elif k >= 128:
                sel_first = is_lo == ((row & (k // 128)) == 0)
            else:
                sel_first = is_lo == ((lane & k) == 0)
            before = (s > bs) | ((s == bs) & (idx < bi))
            s = jnp.where(before == sel_first, s, bs)
            idx = jnp.where(before == sel_first, idx, bi)
            j //= 2
        k *= 2

    out_s_ref[...] = s[:, 0:(_PRE // 128), :].reshape(B, _PRE)
    out_i_ref[...] = idx[:, 0:(_PRE // 128), :].reshape(B, _PRE)


def _run_sort(scores):
    B, N = scores.shape
    return pl.pallas_call(
        _sort_body,
        out_shape=[
            jax.ShapeDtypeStruct((B, _PRE), jnp.float32),
            jax.ShapeDtypeStruct((B, _PRE), jnp.float32),
        ],
    )(scores.reshape(B, N // 128, 128))


def _make_sc_gather(V, D, Btot):
    info = plsc.get_sparse_core_info()
    NC, NS = info.num_cores, info.num_subcores
    NW = NC * NS
    b_per_w = Btot // NW
    nchunk = b_per_w // 128
    mesh = plsc.VectorSubcoreMesh(core_axis_name="c", subcore_axis_name="s")

    @functools.partial(
        pl.kernel, mesh=mesh,
        out_type=jax.ShapeDtypeStruct((Btot, D), jnp.float32),
        scratch_types=[
            pltpu.VMEM((nchunk, 128), jnp.int32),
            pltpu.VMEM((b_per_w, D), jnp.float32),
            pltpu.SemaphoreType.DMA,
        ],
    )
    def k(table_hbm, idx_hbm, out_hbm, idx_v, rows_v, sem):
        wid = lax.axis_index("s") * NC + lax.axis_index("c")
        base = wid * b_per_w
        pltpu.sync_copy(idx_hbm.at[pl.ds(wid * nchunk, nchunk)], idx_v)
        cps = [pltpu.async_copy(table_hbm.at[idx_v.at[j]],
                                rows_v.at[pl.ds(j * 128, 128)], sem)
               for j in range(nchunk)]
        for cp in cps:
            cp.wait()
        pltpu.sync_copy(rows_v, out_hbm.at[pl.ds(base, b_per_w)])

    return k


def _run_gather(table, idx):
    V, D = table.shape
    (Btot,) = idx.shape
    return _make_sc_gather(V, D, Btot)(table, idx.reshape(Btot // 128, 128))


def _fiota(shape, dim):
    return lax.broadcasted_iota(jnp.int32, shape, dim).astype(jnp.float32)


def _argmax_first(v, width):
    m = jnp.max(v, axis=0, keepdims=True)
    iota = _fiota(v.shape, 0)
    idx = jnp.min(jnp.where(v == m, iota, float(width)), axis=0, keepdims=True)
    return idx


def _onehot_gather(v, idx):
    iota = _fiota(v.shape, 0)
    return jnp.sum(jnp.where(iota == idx, v, 0.0), axis=0, keepdims=True)


def _nms_body(scores_ref, regT_ref, obox_ref, oscore_ref):
    B = scores_ref.shape[0]
    K = scores_ref.shape[1]
    nblk = K // _BLK

    box_rows = []
    bevs = []
    for b in range(B):
        reg = regT_ref[b]
        roi_x = reg[76:77]
        roi_y = reg[77:78]
        roi_z = reg[78:79]

        x_bin = _argmax_first(reg[0:12], 12)
        z_bin = _argmax_first(reg[12:24], 12)
        pos_x = x_bin * _LOC_BIN + (_LOC_BIN / 2) - _LOC_SCOPE
        pos_z = z_bin * _LOC_BIN + (_LOC_BIN / 2) - _LOC_SCOPE
        x_res = _onehot_gather(reg[24:36], x_bin) * _LOC_BIN
        z_res = _onehot_gather(reg[36:48], z_bin) * _LOC_BIN
        pos_x = pos_x + x_res
        pos_z = pos_z + z_res
        pos_y = roi_y + reg[48:49]

        ry_bin = _argmax_first(reg[49:61], 12)
        ry_res = _onehot_gather(reg[61:73], ry_bin)
        apc = (2.0 * np.pi) / _NHB
        ry = jnp.mod(ry_bin * apc + ry_res * (apc / 2), 2.0 * np.pi)
        ry = jnp.where(ry > np.pi, ry - 2.0 * np.pi, ry)

        h = reg[73:74] * _MS0 + _MS0
        w = reg[74:75] * _MS1 + _MS1
        l = reg[75:76] * _MS2 + _MS2

        x = pos_x + roi_x
        z = pos_z + roi_z
        y = pos_y + h * 0.5

        box7 = jnp.concatenate([x, y, z, h, w, l, ry], axis=0)
        box_rows.append(box7)

        half_l = l * 0.5
        half_w = w * 0.5
        bevs.append((x - half_l, z - half_w, x + half_l, z + half_w))

    x1 = jnp.concatenate([t[0] for t in bevs], axis=0)
    y1 = jnp.concatenate([t[1] for t in bevs], axis=0)
    x2 = jnp.concatenate([t[2] for t in bevs], axis=0)
    y2 = jnp.concatenate([t[3] for t in bevs], axis=0)
    areas = (x2 - x1) * (y2 - y1)

    def iou_mask(bi, bj):
        s_i = slice(bi * _BLK, (bi + 1) * _BLK)
        s_j = slice(bj * _BLK, (bj + 1) * _BLK)
        x1i = x1[:, s_i][:, :, None]
        y1i = y1[:, s_i][:, :, None]
        x2i = x2[:, s_i][:, :, None]
        y2i = y2[:, s_i][:, :, None]
        ai = areas[:, s_i][:, :, None]
        x1j = x1[:, s_j][:, None, :]
        y1j = y1[:, s_j][:, None, :]
        x2j = x2[:, s_j][:, None, :]
        y2j = y2[:, s_j][:, None, :]
        aj = areas[:, s_j][:, None, :]
        xx1 = jnp.maximum(x1i, x1j)
        yy1 = jnp.maximum(y1i, y1j)
        xx2 = jnp.minimum(x2i, x2j)
        yy2 = jnp.minimum(y2i, y2j)
        inter = jnp.clip(xx2 - xx1, 0.0) * jnp.clip(yy2 - yy1, 0.0)
        iou = inter / jnp.clip(ai + aj - inter, 1e-8)
        return (iou > _THRESH).astype(jnp.float32)

    keep_blocks = [jnp.ones((B, _BLK), dtype=jnp.float32) for _ in range(nblk)]
    tri = (lax.broadcasted_iota(jnp.int32, (1, _BLK, _BLK), 2)
           > lax.broadcasted_iota(jnp.int32, (1, _BLK, _BLK), 1)).astype(jnp.float32)

    for bi in range(nblk):
        mii = iou_mask(bi, bi) * tri
        ext = keep_blocks[bi]

        def w_cond(state):
            return state[1]

        def w_body(state):
            keep_b, _ = state
            sup = jnp.max(keep_b[:, :, None] * mii, axis=1)
            new = ext * (1.0 - sup)
            return new, jnp.any(new != keep_b)

        keep_bi, _ = lax.while_loop(w_cond, w_body, (ext, True))
        keep_blocks[bi] = keep_bi

        for bj in range(bi + 1, nblk):
            mij = iou_mask(bi, bj)
            sup = jnp.max(keep_bi[:, :, None] * mij, axis=1)
            keep_blocks[bj] = keep_blocks[bj] * (1.0 - sup)

    keep = jnp.concatenate(keep_blocks, axis=1)

    keep3 = keep.reshape(B * (K // 128), 128)
    upper_incl = (lax.broadcasted_iota(jnp.int32, (128, 128), 0)
                  <= lax.broadcasted_iota(jnp.int32, (128, 128), 1)).astype(jnp.float32)
    cs = jnp.dot(keep3, upper_incl, preferred_element_type=jnp.float32)
    nrow = K // 128
    row_sums = cs[:, 127:128].reshape(B, nrow)
    upper_strict = (lax.broadcasted_iota(jnp.int32, (nrow, nrow), 0)
                    < lax.broadcasted_iota(jnp.int32, (nrow, nrow), 1)).astype(jnp.float32)
    row_off = jnp.dot(row_sums, upper_strict, preferred_element_type=jnp.float32)
    pos = (cs.reshape(B, nrow, 128) + row_off[:, :, None]).reshape(B, K) - keep

    slot_iota = _fiota((K, _POST), 1)
    for b in range(B):
        oh = jnp.where((pos[b][:, None] == slot_iota) & (keep[b][:, None] > 0.0),
                       1.0, 0.0)
        obox_ref[b] = jnp.dot(box_rows[b], oh, preferred_element_type=jnp.float32,
                              precision=lax.Precision.HIGHEST)
        sc = scores_ref[b].reshape(1, K)
        oscore_ref[b] = jnp.dot(sc, oh, preferred_element_type=jnp.float32,
                                precision=lax.Precision.HIGHEST).reshape(_POST)


def _run(scores_k, regT):
    B, K = scores_k.shape
    obox, oscore = pl.pallas_call(
        _nms_body,
        out_shape=[
            jax.ShapeDtypeStruct((B, 7, _POST), jnp.float32),
            jax.ShapeDtypeStruct((B, _POST), jnp.float32),
        ],
    )(scores_k, regT)
    return obox, oscore


def kernel(rpn_scores, rpn_reg, xyz, gt_boxes3d):
    B, N = rpn_scores.shape
    scores_k, order_f = _run_sort(rpn_scores)
    gidx = order_f.astype(jnp.int32).reshape(B * _PRE)
    table = jnp.concatenate(
        [rpn_reg.reshape(B * N, 76), xyz.reshape(B * N, 3),
         jnp.zeros((B * N, 49), jnp.float32)], axis=1)
    rows = _run_gather(table, gidx)
    regT = jnp.transpose(rows.reshape(B, _PRE, 128), (0, 2, 1))[:, :80, :]
    obox, oscore = _run(scores_k, regT)
    return (jnp.transpose(obox, (0, 2, 1)), oscore)


# speedup vs baseline: 1.0011x; 1.0011x over previous
import functools
import numpy as np

import jax
import jax.numpy as jnp
from jax import lax
from jax.experimental import pallas as pl
from jax.experimental.pallas import tpu as pltpu
from jax.experimental.pallas import tpu_sc as plsc

_MS0 = 1.52563191462
_MS1 = 1.62856739989
_MS2 = 3.88311640418
_LOC_SCOPE = 3.0
_LOC_BIN = 0.5
_NHB = 12
_PRE = 2048
_POST = 512
_THRESH = 0.85
_BLK = 256


def _sort_body(scores_ref, out_s_ref, out_i_ref):
    s = scores_ref[...]
    B = s.shape[0]
    row = lax.broadcasted_iota(jnp.int32, s.shape, 1)
    lane = lax.broadcasted_iota(jnp.int32, s.shape, 2)
    bat = lax.broadcasted_iota(jnp.int32, s.shape, 0)
    n = 128 * 128
    idx = (bat * n + row * 128 + lane).astype(jnp.float32)

    k = 2
    while k <= n:
        j = k // 2
        while j > 0:
            nrow = s.shape[1]
            if j < 128:
                is_lo = (lane[:, :nrow] & j) == 0
                bs = jnp.where(is_lo, jnp.roll(s, -j, axis=2), jnp.roll(s, j, axis=2))
                bi = jnp.where(is_lo, jnp.roll(idx, -j, axis=2), jnp.roll(idx, j, axis=2))
            else:
                m = j // 128
                is_lo = (row[:, :nrow] & m) == 0
                bs = jnp.where(is_lo, jnp.roll(s, -m, axis=1), jnp.roll(s, m, axis=1))
                bi = jnp.where(is_lo, jnp.roll(idx, -m, axis=1), jnp.roll(idx, m, axis=1))
            if k >= n:
                sel_first = is_lo
            elif k >= 128:
                sel_first = is_lo == ((row[:, :nrow] & (k // 128)) == 0)
            else:
                sel_first = is_lo == ((lane[:, :nrow] & k) == 0)
            before = (s > bs) | ((s == bs) & (idx < bi))
            s = jnp.where(before == sel_first, s, bs)
            idx = jnp.where(before == sel_first, idx, bi)
            if k >= n and j >= _PRE:
                keep_rows = max(_PRE // 128, j // 128)
                s = s[:, :keep_rows, :]
                idx = idx[:, :keep_rows, :]
            j //= 2
        k *= 2

    out_s_ref[...] = s[:, 0:(_PRE // 128), :].reshape(B, _PRE)
    out_i_ref[...] = idx[:, 0:(_PRE // 128), :].reshape(B, _PRE)


def _run_sort(scores):
    B, N = scores.shape
    return pl.pallas_call(
        _sort_body,
        out_shape=[
            jax.ShapeDtypeStruct((B, _PRE), jnp.float32),
            jax.ShapeDtypeStruct((B, _PRE), jnp.float32),
        ],
    )(scores.reshape(B, N // 128, 128))


def _make_sc_gather(V, D, Btot):
    info = plsc.get_sparse_core_info()
    NC, NS = info.num_cores, info.num_subcores
    NW = NC * NS
    b_per_w = Btot // NW
    nchunk = b_per_w // 128
    mesh = plsc.VectorSubcoreMesh(core_axis_name="c", subcore_axis_name="s")

    @functools.partial(
        pl.kernel, mesh=mesh,
        out_type=jax.ShapeDtypeStruct((Btot, D), jnp.float32),
        scratch_types=[
            pltpu.VMEM((nchunk, 128), jnp.int32),
            pltpu.VMEM((b_per_w, D), jnp.float32),
            pltpu.SemaphoreType.DMA,
        ],
    )
    def k(table_hbm, idx_hbm, out_hbm, idx_v, rows_v, sem):
        wid = lax.axis_index("s") * NC + lax.axis_index("c")
        base = wid * b_per_w
        pltpu.sync_copy(idx_hbm.at[pl.ds(wid * nchunk, nchunk)], idx_v)
        cps = [pltpu.async_copy(table_hbm.at[idx_v.at[j]],
                                rows_v.at[pl.ds(j * 128, 128)], sem)
               for j in range(nchunk)]
        for cp in cps:
            cp.wait()
        pltpu.sync_copy(rows_v, out_hbm.at[pl.ds(base, b_per_w)])

    return k


def _run_gather(table, idx):
    V, D = table.shape
    (Btot,) = idx.shape
    return _make_sc_gather(V, D, Btot)(table, idx.reshape(Btot // 128, 128))


def _fiota(shape, dim):
    return lax.broadcasted_iota(jnp.int32, shape, dim).astype(jnp.float32)


def _argmax_first(v, width):
    m = jnp.max(v, axis=0, keepdims=True)
    iota = _fiota(v.shape, 0)
    idx = jnp.min(jnp.where(v == m, iota, float(width)), axis=0, keepdims=True)
    return idx


def _onehot_gather(v, idx):
    iota = _fiota(v.shape, 0)
    return jnp.sum(jnp.where(iota == idx, v, 0.0), axis=0, keepdims=True)


def _nms_body(scores_ref, regT_ref, obox_ref, oscore_ref):
    B = scores_ref.shape[0]
    K = scores_ref.shape[1]
    nblk = K // _BLK

    box_rows = []
    bevs = []
    for b in range(B):
        reg = regT_ref[b]
        roi_x = reg[76:77]
        roi_y = reg[77:78]
        roi_z = reg[78:79]

        x_bin = _argmax_first(reg[0:12], 12)
        z_bin = _argmax_first(reg[12:24], 12)
        pos_x = x_bin * _LOC_BIN + (_LOC_BIN / 2) - _LOC_SCOPE
        pos_z = z_bin * _LOC_BIN + (_LOC_BIN / 2) - _LOC_SCOPE
        x_res = _onehot_gather(reg[24:36], x_bin) * _LOC_BIN
        z_res = _onehot_gather(reg[36:48], z_bin) * _LOC_BIN
        pos_x = pos_x + x_res
        pos_z = pos_z + z_res
        pos_y = roi_y + reg[48:49]

        ry_bin = _argmax_first(reg[49:61], 12)
        ry_res = _onehot_gather(reg[61:73], ry_bin)
        apc = (2.0 * np.pi) / _NHB
        ry = jnp.mod(ry_bin * apc + ry_res * (apc / 2), 2.0 * np.pi)
        ry = jnp.where(ry > np.pi, ry - 2.0 * np.pi, ry)

        h = reg[73:74] * _MS0 + _MS0
        w = reg[74:75] * _MS1 + _MS1
        l = reg[75:76] * _MS2 + _MS2

        x = pos_x + roi_x
        z = pos_z + roi_z
        y = pos_y + h * 0.5

        box7 = jnp.concatenate([x, y, z, h, w, l, ry], axis=0)
        box_rows.append(box7)

        half_l = l * 0.5
        half_w = w * 0.5
        bevs.append((x - half_l, z - half_w, x + half_l, z + half_w))

    x1 = jnp.concatenate([t[0] for t in bevs], axis=0)
    y1 = jnp.concatenate([t[1] for t in bevs], axis=0)
    x2 = jnp.concatenate([t[2] for t in bevs], axis=0)
    y2 = jnp.concatenate([t[3] for t in bevs], axis=0)
    areas = (x2 - x1) * (y2 - y1)

    def iou_mask(bi, bj):
        s_i = slice(bi * _BLK, (bi + 1) * _BLK)
        s_j = slice(bj * _BLK, (bj + 1) * _BLK)
        x1i = x1[:, s_i][:, :, None]
        y1i = y1[:, s_i][:, :, None]
        x2i = x2[:, s_i][:, :, None]
        y2i = y2[:, s_i][:, :, None]
        ai = areas[:, s_i][:, :, None]
        x1j = x1[:, s_j][:, None, :]
        y1j = y1[:, s_j][:, None, :]
        x2j = x2[:, s_j][:, None, :]
        y2j = y2[:, s_j][:, None, :]
        aj = areas[:, s_j][:, None, :]
        xx1 = jnp.maximum(x1i, x1j)
        yy1 = jnp.maximum(y1i, y1j)
        xx2 = jnp.minimum(x2i, x2j)
        yy2 = jnp.minimum(y2i, y2j)
        inter = jnp.clip(xx2 - xx1, 0.0) * jnp.clip(yy2 - yy1, 0.0)
        iou = inter / jnp.clip(ai + aj - inter, 1e-8)
        return (iou > _THRESH).astype(jnp.float32)

    keep_blocks = [jnp.ones((B, _BLK), dtype=jnp.float32) for _ in range(nblk)]
    tri = (lax.broadcasted_iota(jnp.int32, (1, _BLK, _BLK), 2)
           > lax.broadcasted_iota(jnp.int32, (1, _BLK, _BLK), 1)).astype(jnp.float32)

    for bi in range(nblk):
        mii = iou_mask(bi, bi) * tri
        ext = keep_blocks[bi]

        def w_cond(state):
            return state[1]

        def w_body(state):
            keep_b, _ = state
            sup = jnp.max(keep_b[:, :, None] * mii, axis=1)
            new = ext * (1.0 - sup)
            return new, jnp.any(new != keep_b)

        keep_bi, _ = lax.while_loop(w_cond, w_body, (ext, True))
        keep_blocks[bi] = keep_bi

        for bj in range(bi + 1, nblk):
            mij = iou_mask(bi, bj)
            sup = jnp.max(keep_bi[:, :, None] * mij, axis=1)
            keep_blocks[bj] = keep_blocks[bj] * (1.0 - sup)

    keep = jnp.concatenate(keep_blocks, axis=1)

    keep3 = keep.reshape(B * (K // 128), 128)
    upper_incl = (lax.broadcasted_iota(jnp.int32, (128, 128), 0)
                  <= lax.broadcasted_iota(jnp.int32, (128, 128), 1)).astype(jnp.float32)
    cs = jnp.dot(keep3, upper_incl, preferred_element_type=jnp.float32)
    nrow = K // 128
    row_sums = cs[:, 127:128].reshape(B, nrow)
    upper_strict = (lax.broadcasted_iota(jnp.int32, (nrow, nrow), 0)
                    < lax.broadcasted_iota(jnp.int32, (nrow, nrow), 1)).astype(jnp.float32)
    row_off = jnp.dot(row_sums, upper_strict, preferred_element_type=jnp.float32)
    pos = (cs.reshape(B, nrow, 128) + row_off[:, :, None]).reshape(B, K) - keep

    slot_iota = _fiota((K, _POST), 1)
    for b in range(B):
        oh = jnp.where((pos[b][:, None] == slot_iota) & (keep[b][:, None] > 0.0),
                       1.0, 0.0)
        obox_ref[b] = jnp.dot(box_rows[b], oh, preferred_element_type=jnp.float32,
                              precision=lax.Precision.HIGHEST)
        sc = scores_ref[b].reshape(1, K)
        oscore_ref[b] = jnp.dot(sc, oh, preferred_element_type=jnp.float32,
                                precision=lax.Precision.HIGHEST).reshape(_POST)


def _run(scores_k, regT):
    B, K = scores_k.shape
    obox, oscore = pl.pallas_call(
        _nms_body,
        out_shape=[
            jax.ShapeDtypeStruct((B, 7, _POST), jnp.float32),
            jax.ShapeDtypeStruct((B, _POST), jnp.float32),
        ],
    )(scores_k, regT)
    return obox, oscore


def kernel(rpn_scores, rpn_reg, xyz, gt_boxes3d):
    B, N = rpn_scores.shape
    scores_k, order_f = _run_sort(rpn_scores)
    gidx = order_f.astype(jnp.int32).reshape(B * _PRE)
    table = jnp.concatenate(
        [rpn_reg.reshape(B * N, 76), xyz.reshape(B * N, 3),
         jnp.zeros((B * N, 49), jnp.float32)], axis=1)
    rows = _run_gather(table, gidx)
    regT = jnp.transpose(rows.reshape(B, _PRE, 128), (0, 2, 1))[:, :80, :]
    obox, oscore = _run(scores_k, regT)
    return (jnp.transpose(obox, (0, 2, 1)), oscore)
